# Initial kernel scaffold; baseline (speedup 1.0000x reference)
import jax
import jax.numpy as jnp
from jax.experimental import pallas as pl


def kernel(h, adj, W, a):
    raise NotImplementedError("write your pallas kernel here")


# speedup vs baseline: 1.3627x; 1.3627x over previous
import jax
import jax.numpy as jnp
from jax.experimental import pallas as pl
from jax.experimental.pallas import tpu as pltpu

N = 4096
F = 128
ALPHA = 0.2
BR = 512
BC = 1024
NR = N // BR
NC = N // BC


def _prologue_kernel(h_ref, w_ref, ht_ref, wt_ref, a1_ref, a2_ref,
                     wh_ref, e1_ref, p1_ref, n1_ref, e2_ref, p2_ref, n2_ref):
    wh = jnp.dot(h_ref[...], w_ref[...], preferred_element_type=jnp.float32)
    wh_ref[...] = wh
    wht = jnp.dot(wt_ref[...], ht_ref[...], preferred_element_type=jnp.float32)
    e1 = jnp.dot(wh, a1_ref[...], preferred_element_type=jnp.float32)
    e2 = jnp.dot(a2_ref[...], wht, preferred_element_type=jnp.float32)
    e1_ref[...] = e1
    p1_ref[...] = jnp.exp(e1)
    n1_ref[...] = jnp.exp(ALPHA * e1)
    e2_ref[...] = e2
    p2_ref[...] = jnp.exp(e2)
    n2_ref[...] = jnp.exp(ALPHA * e2)


def _gat_kernel(adj_ref, wh_ref, e1_ref, p1_ref, n1_ref, e2_ref, p2_ref,
                n2_ref, out_ref, acc_ref, z_ref):
    j = pl.program_id(1)

    @pl.when(j == 0)
    def _():
        acc_ref[...] = jnp.zeros_like(acc_ref)
        z_ref[...] = jnp.zeros_like(z_ref)

    e2 = e2_ref[0:1, pl.ds(j * BC, BC)]
    b2 = p2_ref[0:1, pl.ds(j * BC, BC)]
    d2 = n2_ref[0:1, pl.ds(j * BC, BC)]
    pos = (e1_ref[...] + e2) > 0.0
    es = jnp.where(pos, p1_ref[...], n1_ref[...]) * jnp.where(pos, b2, d2)
    p = 1.0 + adj_ref[...] * (es - 1.0)
    z_ref[...] += jnp.sum(p, axis=1, keepdims=True)
    wh = wh_ref[pl.ds(j * BC, BC), :]
    acc_ref[...] += jnp.dot(p, wh, preferred_element_type=jnp.float32)

    @pl.when(j == NC - 1)
    def _():
        r = acc_ref[...] / z_ref[...]
        out_ref[...] = jnp.where(r > 0.0, r, jnp.exp(r) - 1.0)


def kernel(h, adj, W, a):
    a1 = a[:F].reshape(F, 1)
    a2 = a[F:].reshape(1, F)
    f32 = jnp.float32
    wh, e1, p1, n1, e2, p2, n2 = pl.pallas_call(
        _prologue_kernel,
        out_shape=[
            jax.ShapeDtypeStruct((N, F), f32),
            jax.ShapeDtypeStruct((N, 1), f32),
            jax.ShapeDtypeStruct((N, 1), f32),
            jax.ShapeDtypeStruct((N, 1), f32),
            jax.ShapeDtypeStruct((1, N), f32),
            jax.ShapeDtypeStruct((1, N), f32),
            jax.ShapeDtypeStruct((1, N), f32),
        ],
    )(h, W, h.T, W.T, a1, a2)
    out = pl.pallas_call(
        _gat_kernel,
        grid=(NR, NC),
        in_specs=[
            pl.BlockSpec((BR, BC), lambda i, j: (i, j)),
            pl.BlockSpec((N, F), lambda i, j: (0, 0)),
            pl.BlockSpec((BR, 1), lambda i, j: (i, 0)),
            pl.BlockSpec((BR, 1), lambda i, j: (i, 0)),
            pl.BlockSpec((BR, 1), lambda i, j: (i, 0)),
            pl.BlockSpec((1, N), lambda i, j: (0, 0)),
            pl.BlockSpec((1, N), lambda i, j: (0, 0)),
            pl.BlockSpec((1, N), lambda i, j: (0, 0)),
        ],
        out_specs=pl.BlockSpec((BR, F), lambda i, j: (i, 0)),
        out_shape=jax.ShapeDtypeStruct((N, F), f32),
        scratch_shapes=[
            pltpu.VMEM((BR, F), f32),
            pltpu.VMEM((BR, 1), f32),
        ],
        compiler_params=pltpu.CompilerParams(
            dimension_semantics=("parallel", "arbitrary")),
    )(adj, wh, e1, p1, n1, e2, p2, n2)
    return out


# speedup vs baseline: 1.4486x; 1.0630x over previous
import jax
import jax.numpy as jnp
from jax.experimental import pallas as pl
from jax.experimental.pallas import tpu as pltpu

N = 4096
F = 128
ALPHA = 0.2
BR = 512
BC = 1024
NR = N // BR
NC = N // BC


def _prologue_kernel(h_ref, w_ref, ht_ref, wt_ref, a1_ref, a2_ref,
                     wh_ref, p1_ref, n1_ref, p2_ref, n2_ref, s_ref):
    wh = jnp.dot(h_ref[...], w_ref[...], preferred_element_type=jnp.float32)
    wh_ref[...] = wh
    wht = jnp.dot(wt_ref[...], ht_ref[...], preferred_element_type=jnp.float32)
    e1 = jnp.dot(wh, a1_ref[...], preferred_element_type=jnp.float32)
    e2 = jnp.dot(a2_ref[...], wht, preferred_element_type=jnp.float32)
    p1_ref[...] = jnp.exp(e1)
    n1_ref[...] = jnp.exp(ALPHA * e1)
    p2_ref[...] = jnp.exp(e2)
    n2_ref[...] = jnp.exp(ALPHA * e2)
    s_ref[...] = jnp.sum(wh, axis=0, keepdims=True)


def _gat_kernel(adj_ref, wh_ref, p1_ref, n1_ref, p2_ref, n2_ref, s_ref,
                out_ref, acc_ref, z_ref):
    j = pl.program_id(1)

    @pl.when(j == 0)
    def _():
        acc_ref[...] = jnp.zeros_like(acc_ref)
        z_ref[...] = jnp.zeros_like(z_ref)

    b2 = p2_ref[0:1, pl.ds(j * BC, BC)]
    d2 = n2_ref[0:1, pl.ds(j * BC, BC)]
    adj = adj_ref[...]
    es = jnp.maximum(p1_ref[...] * b2, n1_ref[...] * d2)
    q = adj * es - adj
    z_ref[...] += jnp.sum(q, axis=1, keepdims=True)
    wh = wh_ref[pl.ds(j * BC, BC), :]
    acc_ref[...] += jnp.dot(q, wh, preferred_element_type=jnp.float32)

    @pl.when(j == NC - 1)
    def _():
        r = (acc_ref[...] + s_ref[...]) / (z_ref[...] + float(N))
        out_ref[...] = jnp.where(r > 0.0, r, jnp.exp(r) - 1.0)


def kernel(h, adj, W, a):
    a1 = a[:F].reshape(F, 1)
    a2 = a[F:].reshape(1, F)
    f32 = jnp.float32
    wh, p1, n1, p2, n2, s = pl.pallas_call(
        _prologue_kernel,
        out_shape=[
            jax.ShapeDtypeStruct((N, F), f32),
            jax.ShapeDtypeStruct((N, 1), f32),
            jax.ShapeDtypeStruct((N, 1), f32),
            jax.ShapeDtypeStruct((1, N), f32),
            jax.ShapeDtypeStruct((1, N), f32),
            jax.ShapeDtypeStruct((1, F), f32),
        ],
    )(h, W, h.T, W.T, a1, a2)
    out = pl.pallas_call(
        _gat_kernel,
        grid=(NR, NC),
        in_specs=[
            pl.BlockSpec((BR, BC), lambda i, j: (i, j)),
            pl.BlockSpec((N, F), lambda i, j: (0, 0)),
            pl.BlockSpec((BR, 1), lambda i, j: (i, 0)),
            pl.BlockSpec((BR, 1), lambda i, j: (i, 0)),
            pl.BlockSpec((1, N), lambda i, j: (0, 0)),
            pl.BlockSpec((1, N), lambda i, j: (0, 0)),
            pl.BlockSpec((1, F), lambda i, j: (0, 0)),
        ],
        out_specs=pl.BlockSpec((BR, F), lambda i, j: (i, 0)),
        out_shape=jax.ShapeDtypeStruct((N, F), f32),
        scratch_shapes=[
            pltpu.VMEM((BR, F), f32),
            pltpu.VMEM((BR, 1), f32),
        ],
        compiler_params=pltpu.CompilerParams(
            dimension_semantics=("parallel", "arbitrary")),
    )(adj, wh, p1, n1, p2, n2, s)
    return out


# speedup vs baseline: 1.8124x; 1.2512x over previous
import jax
import jax.numpy as jnp
from jax.experimental import pallas as pl
from jax.experimental.pallas import tpu as pltpu

N = 4096
F = 128
ALPHA = 0.2
BR = 512
NR = N // BR


def _prologue_kernel(h_ref, w_ref, ht_ref, wt_ref, a1_ref, a2_ref,
                     wh_ref, p1_ref, n1_ref, p2_ref, n2_ref, s_ref):
    wh = jnp.dot(h_ref[...], w_ref[...], preferred_element_type=jnp.float32)
    wh_ref[...] = wh
    wht = jnp.dot(wt_ref[...], ht_ref[...], preferred_element_type=jnp.float32)
    e1 = jnp.dot(wh, a1_ref[...], preferred_element_type=jnp.float32)
    e2 = jnp.dot(a2_ref[...], wht, preferred_element_type=jnp.float32)
    p1_ref[...] = jnp.exp(e1)
    n1_ref[...] = jnp.exp(ALPHA * e1)
    p2_ref[...] = jnp.exp(e2)
    n2_ref[...] = jnp.exp(ALPHA * e2)
    s_ref[...] = jnp.sum(wh, axis=0, keepdims=True)


def _gat_kernel(adj_ref, wh_ref, p1_ref, n1_ref, p2_ref, n2_ref, s_ref,
                out_ref):
    adj = adj_ref[...]
    es = jnp.maximum(p1_ref[...] * p2_ref[...], n1_ref[...] * n2_ref[...])
    q = adj * es - adj
    z = jnp.sum(q, axis=1, keepdims=True) + float(N)
    num = jnp.dot(q, wh_ref[...], preferred_element_type=jnp.float32)
    r = (num + s_ref[...]) / z
    out_ref[...] = jnp.where(r > 0.0, r, jnp.exp(r) - 1.0)


def kernel(h, adj, W, a):
    a1 = a[:F].reshape(F, 1)
    a2 = a[F:].reshape(1, F)
    f32 = jnp.float32
    wh, p1, n1, p2, n2, s = pl.pallas_call(
        _prologue_kernel,
        out_shape=[
            jax.ShapeDtypeStruct((N, F), f32),
            jax.ShapeDtypeStruct((N, 1), f32),
            jax.ShapeDtypeStruct((N, 1), f32),
            jax.ShapeDtypeStruct((1, N), f32),
            jax.ShapeDtypeStruct((1, N), f32),
            jax.ShapeDtypeStruct((1, F), f32),
        ],
    )(h, W, h.T, W.T, a1, a2)
    out = pl.pallas_call(
        _gat_kernel,
        grid=(NR,),
        in_specs=[
            pl.BlockSpec((BR, N), lambda i: (i, 0)),
            pl.BlockSpec((N, F), lambda i: (0, 0)),
            pl.BlockSpec((BR, 1), lambda i: (i, 0)),
            pl.BlockSpec((BR, 1), lambda i: (i, 0)),
            pl.BlockSpec((1, N), lambda i: (0, 0)),
            pl.BlockSpec((1, N), lambda i: (0, 0)),
            pl.BlockSpec((1, F), lambda i: (0, 0)),
        ],
        out_specs=pl.BlockSpec((BR, F), lambda i: (i, 0)),
        out_shape=jax.ShapeDtypeStruct((N, F), f32),
        compiler_params=pltpu.CompilerParams(
            dimension_semantics=("arbitrary",)),
    )(adj, wh, p1, n1, p2, n2, s)
    return out
